# Initial kernel scaffold; baseline (speedup 1.0000x reference)
import jax
import jax.numpy as jnp
from jax.experimental import pallas as pl


def kernel(x_input, states, tensions, up_W, up_b, down_W, down_b):
    raise NotImplementedError("write your pallas kernel here")


# speedup vs baseline: 1.1849x; 1.1849x over previous
import functools

import jax
import jax.numpy as jnp
from jax.experimental import pallas as pl
from jax.experimental.pallas import tpu as pltpu

N_EXPERTS = 8
CELLS_PER = 4
CELL_DIM = 2048
HIDDEN = 2048
VOCAB = 4096
TOP_K = 2

BH = 512
H_TILES = (2 * HIDDEN) // BH


def _moe_body(idx_ref, vals_ref, states_ref, upw_ref, upb_ref, dnw_ref,
              dnb_ref, out_ref, c_scr):
    k = pl.program_id(0)
    t = pl.program_id(1)

    @pl.when((k == 0) & (t == 0))
    def _init():
        c_scr[...] = jnp.mean(states_ref[...], axis=0, keepdims=True)
        out_ref[...] = jnp.zeros_like(out_ref)

    w = vals_ref[k]
    c = c_scr[...]
    pre = jnp.dot(c, upw_ref[0], preferred_element_type=jnp.float32)
    pre = pre + upb_ref[0]
    h = 0.5 * pre * (1.0 + jax.lax.erf(pre * (2.0 ** -0.5)))
    part = jnp.dot(h, dnw_ref[0], preferred_element_type=jnp.float32)

    @pl.when(t == 0)
    def _bias():
        out_ref[...] += w * dnb_ref[0]

    out_ref[...] += w * part


def _moe_call(topk_idx, topk_vals, states, up_W, up_b, down_W, down_b):
    grid_spec = pltpu.PrefetchScalarGridSpec(
        num_scalar_prefetch=2,
        grid=(TOP_K, H_TILES),
        in_specs=[
            pl.BlockSpec((CELLS_PER * N_EXPERTS, HIDDEN),
                         lambda k, t, idx, vals: (0, 0)),
            pl.BlockSpec((1, HIDDEN, BH),
                         lambda k, t, idx, vals: (idx[k], 0, t)),
            pl.BlockSpec((1, 1, BH),
                         lambda k, t, idx, vals: (idx[k], 0, t)),
            pl.BlockSpec((1, BH, VOCAB),
                         lambda k, t, idx, vals: (idx[k], t, 0)),
            pl.BlockSpec((1, 1, VOCAB),
                         lambda k, t, idx, vals: (idx[k], 0, 0)),
        ],
        out_specs=pl.BlockSpec((1, VOCAB), lambda k, t, idx, vals: (0, 0)),
        scratch_shapes=[pltpu.VMEM((1, HIDDEN), jnp.float32)],
    )
    return pl.pallas_call(
        _moe_body,
        grid_spec=grid_spec,
        out_shape=jax.ShapeDtypeStruct((1, VOCAB), jnp.float32),
    )(topk_idx, topk_vals, states, up_W, up_b, down_W, down_b)


def kernel(x_input, states, tensions, up_W, up_b, down_W, down_b):
    expert_tensions = tensions.reshape(N_EXPERTS, CELLS_PER).mean(axis=-1)
    weights = jax.nn.softmax(expert_tensions / 0.1, axis=-1)
    topk_vals, topk_idx = jax.lax.top_k(weights, TOP_K)
    topk_vals = topk_vals / jnp.sum(topk_vals)
    out = _moe_call(topk_idx.astype(jnp.int32), topk_vals, states,
                    up_W, up_b.reshape(N_EXPERTS, 1, 2 * HIDDEN),
                    down_W, down_b.reshape(N_EXPERTS, 1, VOCAB))
    phi = jnp.zeros((), dtype=jnp.float32)
    return (out.reshape(VOCAB), phi, weights)
